# baseline (speedup 1.0000x reference)
import functools

import jax
import jax.numpy as jnp
from jax import lax
from jax.experimental import pallas as pl
from jax.experimental.pallas import tpu as pltpu
from jax.experimental.pallas import tpu_sc as plsc

_B = 1024
_L = 200
_D = 64
_DP = 128
_NC = 2
_NS = 16
_NW = _NC * _NS
_N = _B // _NW
_SPLIT = 104
_R = 3
_RO = 2


def _make_embed():
    mesh = plsc.VectorSubcoreMesh(core_axis_name="c", subcore_axis_name="s")

    @functools.partial(
        pl.kernel,
        mesh=mesh,
        out_type=jax.ShapeDtypeStruct((_B * _L // 2, _DP), jnp.float32),
        compiler_params=pltpu.CompilerParams(use_tc_tiling_on_sc=False),
        scratch_types=[
            pltpu.VMEM((_R * _L,), jnp.int32),
            pltpu.VMEM((_R, _L, _DP), jnp.float32),
            pltpu.VMEM((_RO, _L // 2, _DP), jnp.float32),
            pltpu.VMEM((_L, _D), jnp.float32),
            pltpu.SemaphoreType.DMA((_R,)),
            pltpu.SemaphoreType.DMA((_R,)),
            pltpu.SemaphoreType.DMA((_RO,)),
        ],
    )
    def embed(x_hbm, tok_hbm, pos_hbm, out_hbm, idx_v, rows_v, obuf_v, pos_v,
              p_sem, g_sem, w_sem):
        s = lax.axis_index("s")
        c = lax.axis_index("c")
        wid = s * _NC + c
        base = wid * _N

        pltpu.sync_copy(pos_hbm.at[:, pl.ds(0, _D)], pos_v)

        def wb_copy(i, ob):
            return pltpu.make_async_copy(
                obuf_v.at[ob],
                out_hbm.at[pl.ds((base + i) * (_L // 2), _L // 2)],
                w_sem.at[ob])

        def idx_copy(i, b):
            return pltpu.make_async_copy(
                x_hbm.at[pl.ds((base + i) * _L, _L)],
                idx_v.at[pl.ds(b * _L, _L)],
                p_sem.at[b])

        def start_gathers(b):
            pltpu.async_copy(
                tok_hbm.at[idx_v.at[pl.ds(b * _L, _SPLIT)]],
                rows_v.at[b, pl.ds(0, _SPLIT)],
                g_sem.at[b])
            pltpu.async_copy(
                tok_hbm.at[idx_v.at[pl.ds(b * _L + _SPLIT, _L - _SPLIT)]],
                rows_v.at[b, pl.ds(_SPLIT, _L - _SPLIT)],
                g_sem.at[b])

        def wait_gathers(b):
            pltpu.make_async_copy(
                tok_hbm.at[idx_v.at[pl.ds(b * _L, _SPLIT)]],
                rows_v.at[b, pl.ds(0, _SPLIT)],
                g_sem.at[b]).wait()
            pltpu.make_async_copy(
                tok_hbm.at[idx_v.at[pl.ds(b * _L + _SPLIT, _L - _SPLIT)]],
                rows_v.at[b, pl.ds(_SPLIT, _L - _SPLIT)],
                g_sem.at[b]).wait()

        def prefetch(i, b):
            idx_copy(i, b).start()

        def launch(i, b):
            idx_copy(i, b).wait()
            start_gathers(b)

        def finish(i, b, ob, guard):
            wait_gathers(b)
            if guard:
                wb_copy(i - _RO, ob).wait()

            def sel_body(j, carry):
                r2 = j >> 1
                h = (j & 1) * _D
                for col in range(_D // 16):
                    sl = pl.ds(col * 16, 16)
                    osl = pl.ds(h + col * 16, 16)
                    obuf_v[ob, r2, osl] = rows_v[b, j, sl] + pos_v[j, sl]
                return carry

            lax.fori_loop(0, _L, sel_body, 0)
            wb_copy(i, ob).start()

        prefetch(0, 0)
        prefetch(1, 1)
        launch(0, 0)
        for i in range(_N):
            if i + 1 < _N:
                launch(i + 1, (i + 1) % _R)
            finish(i, i % _R, i % _RO, guard=(i >= _RO))
            if i + 2 < _N:
                prefetch(i + 2, (i + 2) % _R)
        for i in range(_N - _RO, _N):
            wb_copy(i, i % _RO).wait()

    return embed


_embed = _make_embed()


def kernel(x, token_table, pos_table):
    x_flat = x.reshape(-1).astype(jnp.int32)
    tok_pad = jnp.pad(token_table, ((0, 0), (0, _DP - _D)))
    pos_pad = jnp.pad(pos_table[:_L], ((0, 0), (0, _DP - _D)))
    out2 = _embed(x_flat, tok_pad, pos_pad)
    return out2.reshape(_B, _L, _D)


# speedup vs baseline: 1.4307x; 1.4307x over previous
import functools

import jax
import jax.numpy as jnp
from jax import lax
from jax.experimental import pallas as pl
from jax.experimental.pallas import tpu as pltpu
from jax.experimental.pallas import tpu_sc as plsc

_B = 1024
_L = 200
_D = 64
_NC = 2
_NS = 16
_NW = _NC * _NS
_N = _B // _NW
_SPLIT = 104
_R = 6


def _make_embed():
    mesh = plsc.VectorSubcoreMesh(core_axis_name="c", subcore_axis_name="s")

    @functools.partial(
        pl.kernel,
        mesh=mesh,
        out_type=jax.ShapeDtypeStruct((_B, _L, _D), jnp.float32),
        compiler_params=pltpu.CompilerParams(use_tc_tiling_on_sc=False),
        scratch_types=[
            pltpu.VMEM((_R, _L), jnp.int32),
            pltpu.VMEM((_R, _L, _D), jnp.float32),
            pltpu.VMEM((_L, _D), jnp.float32),
            pltpu.SemaphoreType.DMA((_R,)),
            pltpu.SemaphoreType.DMA((_R,)),
            pltpu.SemaphoreType.DMA((_R,)),
        ],
    )
    def embed(x_hbm, tok_hbm, pos_hbm, out_hbm, idx_v, rows_v, pos_v,
              p_sem, g_sem, w_sem):
        s = lax.axis_index("s")
        c = lax.axis_index("c")
        wid = s * _NC + c
        base = wid * _N

        pltpu.sync_copy(pos_hbm.at[pl.ds(0, _L)], pos_v)

        def wb_copy(i, b):
            return pltpu.make_async_copy(
                rows_v.at[b], out_hbm.at[base + i], w_sem.at[b])

        def prefill_copies(i, b):
            return (
                pltpu.make_async_copy(
                    x_hbm.at[base + i], idx_v.at[b], p_sem.at[b]),
            )

        def start_gathers(b):
            pltpu.async_copy(
                tok_hbm.at[idx_v.at[b]], rows_v.at[b], g_sem.at[b])

        def wait_gathers(b):
            pltpu.make_async_copy(
                tok_hbm.at[idx_v.at[b]], rows_v.at[b], g_sem.at[b]).wait()

        def prefetch(i, b, guard):
            if guard:
                wb_copy(i - _R, b).wait()
            (cpi,) = prefill_copies(i, b)
            cpi.start()

        def launch(i, b):
            (cpi,) = prefill_copies(i, b)
            cpi.wait()
            start_gathers(b)

        def finish(i, b):
            wait_gathers(b)

            def add_body(r, carry):
                for col in range(_D // 16):
                    sl = pl.ds(col * 16, 16)
                    rows_v[b, r, sl] = rows_v[b, r, sl] + pos_v[r, sl]
                return carry

            lax.fori_loop(0, _L, add_body, 0)
            wb_copy(i, b).start()

        for r in range(4):
            prefetch(r, r % _R, guard=False)
        launch(0, 0)
        launch(1, 1)
        for i in range(_N):
            if i + 2 < _N:
                launch(i + 2, (i + 2) % _R)
            finish(i, i % _R)
            if i + 4 < _N:
                prefetch(i + 4, (i + 4) % _R, guard=(i + 4 >= _R))
        for i in range(_N - _R, _N):
            wb_copy(i, i % _R).wait()

    return embed


_embed = _make_embed()


def kernel(x, token_table, pos_table):
    return _embed(x.astype(jnp.int32), token_table, pos_table)
